# baseline (speedup 1.0000x reference)
import functools

import jax
import jax.numpy as jnp
from jax import lax
from jax.experimental import pallas as pl
from jax.experimental.pallas import tpu as pltpu
from jax.experimental.pallas import tpu_sc as plsc

KNB = 20
KPAD = 32
NBATCH = 4
NPTS = 1024
ROWT = 256

_F32 = jnp.float32
_BF16 = jnp.bfloat16


def _t1_body(xr_ref, xt_ref, idx_ref):
    b = pl.program_id(0)
    xr = xr_ref[0]
    xt = xt_ref[0]
    d2r = jnp.sum(xr * xr, axis=1, keepdims=True)
    d2c = jnp.sum(xt * xt, axis=0, keepdims=True)
    g = lax.dot_general(xr.astype(_BF16), xt.astype(_BF16),
                        (((1,), (0,)), ((), ())),
                        preferred_element_type=_F32)
    dist = d2r + d2c - 2.0 * g
    r, n = dist.shape
    cols = lax.broadcasted_iota(jnp.int32, (r, n), 1)
    slot = lax.broadcasted_iota(jnp.int32, (r, KPAD), 1)

    def body(kk, carry):
        d, acc = carry
        j = jnp.argmin(d, axis=1).astype(jnp.int32)[:, None]
        d = jnp.where(cols == j, jnp.inf, d)
        acc = jnp.where(slot == kk, j, acc)
        return d, acc

    _, acc = lax.fori_loop(0, KNB, body,
                           (dist, jnp.zeros((r, KPAD), jnp.int32)))
    idx_ref[...] = acc[:, :KNB] + b * n


def _t1(xp, xt):
    bq, nq, d = xp.shape
    nr = nq // ROWT
    return pl.pallas_call(
        _t1_body,
        grid=(bq, nr),
        in_specs=[
            pl.BlockSpec((1, ROWT, d), lambda b, r: (b, r, 0)),
            pl.BlockSpec((1, d, nq), lambda b, r: (b, 0, 0)),
        ],
        out_specs=pl.BlockSpec((ROWT, KNB),
                               lambda b, r, _n=nr: (b * _n + r, 0)),
        out_shape=jax.ShapeDtypeStruct((bq * nq, KNB), jnp.int32),
    )(xp, xt)


def _sc_gather(table, idxe):
    nrows = idxe.shape[0]
    dp = table.shape[1]
    nw = 32
    rpw = nrows // nw
    grp = 128
    ngrp = rpw // grp
    sds = jax.ShapeDtypeStruct((nrows, dp), _F32)
    mesh = plsc.VectorSubcoreMesh(core_axis_name="c", subcore_axis_name="s")

    @functools.partial(
        pl.kernel,
        out_type=sds,
        mesh=mesh,
        scratch_types=[
            pltpu.VMEM((rpw,), jnp.int32),
            pltpu.VMEM((grp, dp), _F32),
            pltpu.VMEM((grp, dp), _F32),
            pltpu.SemaphoreType.DMA,
            pltpu.SemaphoreType.DMA,
            pltpu.SemaphoreType.DMA,
            pltpu.SemaphoreType.DMA,
        ],
        compiler_params=pltpu.CompilerParams(use_tc_tiling_on_sc=False),
    )
    def k(table_h, idx_h, out_h, idxv, buf0, buf1, gs0, gs1, ws0, ws1):
        wid = lax.axis_index("s") * 2 + lax.axis_index("c")
        base = wid * rpw
        pltpu.sync_copy(idx_h.at[pl.ds(base, rpw)], idxv)
        pltpu.async_copy(table_h.at[idxv.at[pl.ds(0, grp)]], buf0, gs0)
        bufs = ((buf0, gs0, ws0), (buf1, gs1, ws1))

        def do_group(gg, s):
            buf, gs, ws = bufs[s]
            obuf, ogs, ows = bufs[1 - s]

            @pl.when(gg >= 1)
            def _():
                pltpu.make_async_copy(
                    obuf, out_h.at[pl.ds(base + (gg - 1) * grp, grp)],
                    ows).wait()

            @pl.when(gg + 1 < ngrp)
            def _():
                pltpu.async_copy(
                    table_h.at[idxv.at[pl.ds((gg + 1) * grp, grp)]],
                    obuf, ogs)

            pltpu.make_async_copy(
                table_h.at[idxv.at[pl.ds(gg * grp, grp)]], buf, gs).wait()
            pltpu.async_copy(buf, out_h.at[pl.ds(base + gg * grp, grp)], ws)

        def pair(t, _):
            do_group(2 * t, 0)
            do_group(2 * t + 1, 1)
            return 0

        lax.fori_loop(0, ngrp // 2, pair, 0)
        lbuf, _, lws = bufs[(ngrp - 1) % 2]
        pltpu.make_async_copy(
            lbuf, out_h.at[pl.ds(base + (ngrp - 1) * grp, grp)], lws).wait()

    return k(table, idxe)


def _edge_body(x_ref, xg_ref, wa_ref, wb_ref,
               fmax_ref, cs_ref, cq_ref, acc_s, acc_q):
    t = pl.program_id(0)
    nt = pl.num_programs(0)

    @pl.when(t == 0)
    def _():
        acc_s[...] = jnp.zeros_like(acc_s)
        acc_q[...] = jnp.zeros_like(acc_q)

    xi = x_ref[...]
    dn = (((1,), (0,)), ((), ()))
    fb = lax.dot_general(xi.astype(_BF16), wb_ref[...], dn,
                         preferred_element_type=_F32)
    fmax = None
    fsum = None
    fsq = None
    for kk in range(KNB):
        e1 = (xg_ref[kk] - xi).astype(_BF16)
        fk = lax.dot_general(e1, wa_ref[...], dn,
                             preferred_element_type=_F32) + fb
        if kk == 0:
            fmax, fsum, fsq = fk, fk, fk * fk
        else:
            fmax = jnp.maximum(fmax, fk)
            fsum = fsum + fk
            fsq = fsq + fk * fk
    fmax_ref[...] = fmax
    acc_s[...] += jnp.sum(fsum, axis=0, keepdims=True)
    acc_q[...] += jnp.sum(fsq, axis=0, keepdims=True)

    @pl.when(t == nt - 1)
    def _():
        cs_ref[...] = acc_s[...]
        cq_ref[...] = acc_q[...]


def _edge(x2d, xg3d, wa_bf, wb_bf):
    bn, d = x2d.shape
    c = wa_bf.shape[1]
    nt = bn // ROWT
    return pl.pallas_call(
        _edge_body,
        grid=(nt,),
        in_specs=[
            pl.BlockSpec((ROWT, d), lambda t: (t, 0)),
            pl.BlockSpec((KNB, ROWT, d), lambda t: (0, t, 0)),
            pl.BlockSpec((d, c), lambda t: (0, 0)),
            pl.BlockSpec((d, c), lambda t: (0, 0)),
        ],
        out_specs=[
            pl.BlockSpec((ROWT, c), lambda t: (t, 0)),
            pl.BlockSpec((1, c), lambda t: (0, 0)),
            pl.BlockSpec((1, c), lambda t: (0, 0)),
        ],
        out_shape=[
            jax.ShapeDtypeStruct((bn, c), _F32),
            jax.ShapeDtypeStruct((1, c), _F32),
            jax.ShapeDtypeStruct((1, c), _F32),
        ],
        scratch_shapes=[pltpu.VMEM((1, c), _F32), pltpu.VMEM((1, c), _F32)],
    )(x2d, xg3d, wa_bf, wb_bf)


def _bn_body(fmax_ref, cs_ref, cq_ref, g_ref, b_ref, o_ref):
    m = float(fmax_ref.shape[0] * KNB)
    mean = cs_ref[...] / m
    var = cq_ref[...] / m - mean * mean
    inv = lax.rsqrt(var + 1e-5)
    y = (fmax_ref[...] - mean) * inv * g_ref[...] + b_ref[...]
    o_ref[...] = jnp.where(y >= 0, y, 0.2 * y)


def _bn(fmax, cs, cq, gamma, beta):
    bn, c = fmax.shape
    return pl.pallas_call(
        _bn_body,
        out_shape=jax.ShapeDtypeStruct((bn, c), _F32),
    )(fmax, cs, cq, gamma, beta)


def _t3_body(x1_ref, x2_ref, x3_ref, x4_ref, w1_ref, w2_ref, w3_ref, w4_ref,
             g_ref, b_ref, o_ref, cs_ref, cq_ref):
    bb = pl.program_id(0)
    dn = (((1,), (0,)), ((), ()))
    y = (lax.dot_general(x1_ref[...].astype(_BF16), w1_ref[...], dn,
                         preferred_element_type=_F32)
         + lax.dot_general(x2_ref[...].astype(_BF16), w2_ref[...], dn,
                           preferred_element_type=_F32)
         + lax.dot_general(x3_ref[...].astype(_BF16), w3_ref[...], dn,
                           preferred_element_type=_F32)
         + lax.dot_general(x4_ref[...].astype(_BF16), w4_ref[...], dn,
                           preferred_element_type=_F32))

    @pl.when(bb == 0)
    def _():
        cs_ref[...] = jnp.zeros_like(cs_ref)
        cq_ref[...] = jnp.zeros_like(cq_ref)

    cs_ref[...] += jnp.sum(y, axis=0, keepdims=True)
    cq_ref[...] += jnp.sum(y * y, axis=0, keepdims=True)
    o_ref[pl.ds(bb, 1), :] = jnp.max(y, axis=0, keepdims=True)

    @pl.when(bb == NBATCH - 1)
    def _():
        m = float(NBATCH * NPTS)
        mean = cs_ref[...] / m
        var = cq_ref[...] / m - mean * mean
        inv = lax.rsqrt(var + 1e-5)
        v = (o_ref[...] - mean) * inv * g_ref[...] + b_ref[...]
        o_ref[...] = jnp.where(v >= 0, v, 0.2 * v)


def _t3(x1, x2, x3, x4, w_parts, gamma, beta):
    w1t, w2t, w3t, w4t = w_parts
    cdim = w1t.shape[1]
    specs = []
    for xin in (x1, x2, x3, x4):
        cc = xin.shape[1]
        specs.append(pl.BlockSpec((NPTS, cc), lambda b: (b, 0)))
    for wt in (w1t, w2t, w3t, w4t):
        dd = wt.shape[0]
        specs.append(pl.BlockSpec((dd, cdim), lambda b: (0, 0)))
    specs.append(pl.BlockSpec((1, cdim), lambda b: (0, 0)))
    specs.append(pl.BlockSpec((1, cdim), lambda b: (0, 0)))
    return pl.pallas_call(
        _t3_body,
        grid=(NBATCH,),
        in_specs=specs,
        out_specs=pl.BlockSpec((NBATCH, cdim), lambda b: (0, 0)),
        out_shape=jax.ShapeDtypeStruct((NBATCH, cdim), _F32),
        scratch_shapes=[pltpu.VMEM((1, cdim), _F32),
                        pltpu.VMEM((1, cdim), _F32)],
    )(x1, x2, x3, x4, w1t, w2t, w3t, w4t, gamma, beta)


def _run_block(x3d, w, gamma, beta):
    bq, nq, d = x3d.shape
    dp = 16 if d < 16 else d
    if dp != d:
        x3d = jnp.pad(x3d, ((0, 0), (0, 0), (0, dp - d)))
    xt = jnp.transpose(x3d, (0, 2, 1))
    idx2 = _t1(x3d, xt)
    idxk = jnp.transpose(idx2, (1, 0)).reshape(-1)
    x2d = x3d.reshape(bq * nq, dp)
    xg = _sc_gather(x2d, idxk)
    wa_bf = jnp.pad(w[:, :d].T.astype(_BF16), ((0, dp - d), (0, 0)))
    wb_bf = jnp.pad(w[:, d:].T.astype(_BF16), ((0, dp - d), (0, 0)))
    fmax, cs, cq = _edge(x2d, xg.reshape(KNB, bq * nq, dp), wa_bf, wb_bf)
    return _bn(fmax, cs, cq, gamma.reshape(1, -1), beta.reshape(1, -1))


def kernel(x, fc1_w, fc2_w, fc3_w, fc4_w, fc5_w,
           bn1_g, bn1_b, bn2_g, bn2_b, bn3_g, bn3_b,
           bn4_g, bn4_b, bn5_g, bn5_b):
    bq, nq, _ = x.shape
    x1 = _run_block(x, fc1_w, bn1_g, bn1_b)
    x2 = _run_block(x1.reshape(bq, nq, -1), fc2_w, bn2_g, bn2_b)
    x3 = _run_block(x2.reshape(bq, nq, -1), fc3_w, bn3_g, bn3_b)
    x4 = _run_block(x3.reshape(bq, nq, -1), fc4_w, bn4_g, bn4_b)
    w5_parts = (fc5_w[:, :64].T.astype(_BF16),
                fc5_w[:, 64:128].T.astype(_BF16),
                fc5_w[:, 128:256].T.astype(_BF16),
                fc5_w[:, 256:].T.astype(_BF16))
    return _t3(x1, x2, x3, x4, w5_parts,
               bn5_g.reshape(1, -1), bn5_b.reshape(1, -1))


# speedup vs baseline: 1.3863x; 1.3863x over previous
import functools

import jax
import jax.numpy as jnp
from jax import lax
from jax.experimental import pallas as pl
from jax.experimental.pallas import tpu as pltpu
from jax.experimental.pallas import tpu_sc as plsc

KNB = 20
KPAD = 32
NBATCH = 4
NPTS = 1024
ROWT = 256

_F32 = jnp.float32
_BF16 = jnp.bfloat16


def _t1_body(xr_ref, xt_ref, idx_ref):
    b = pl.program_id(0)
    xr = xr_ref[0]
    xt = xt_ref[0]
    d2r = jnp.sum(xr * xr, axis=1, keepdims=True)
    d2c = jnp.sum(xt * xt, axis=0, keepdims=True)
    g = lax.dot_general(xr.astype(_BF16), xt.astype(_BF16),
                        (((1,), (0,)), ((), ())),
                        preferred_element_type=_F32)
    dist = d2r + d2c - 2.0 * g
    r, n = dist.shape
    cols = lax.broadcasted_iota(jnp.int32, (r, n), 1)
    slot = lax.broadcasted_iota(jnp.int32, (r, KPAD), 1)

    def body(kk, carry):
        d, acc = carry
        m = jnp.min(d, axis=1, keepdims=True)
        j = jnp.min(jnp.where(d == m, cols, n), axis=1, keepdims=True)
        d = jnp.where(cols == j, jnp.inf, d)
        acc = jnp.where(slot == kk, j, acc)
        return d, acc

    _, acc = lax.fori_loop(0, KNB, body,
                           (dist, jnp.zeros((r, KPAD), jnp.int32)))
    idx_ref[...] = acc[:, :KNB] + b * n


def _t1(xp, xt, rowt=1024):
    bq, nq, d = xp.shape
    nr = nq // rowt
    return pl.pallas_call(
        _t1_body,
        grid=(bq, nr),
        in_specs=[
            pl.BlockSpec((1, rowt, d), lambda b, r: (b, r, 0)),
            pl.BlockSpec((1, d, nq), lambda b, r: (b, 0, 0)),
        ],
        out_specs=pl.BlockSpec((rowt, KNB),
                               lambda b, r, _n=nr: (b * _n + r, 0)),
        out_shape=jax.ShapeDtypeStruct((bq * nq, KNB), jnp.int32),
    )(xp, xt)


def _sc_gather(table, idxe):
    nrows = idxe.shape[0]
    dp = table.shape[1]
    nw = 32
    rpw = nrows // nw
    grp = 128
    ngrp = rpw // grp
    sds = jax.ShapeDtypeStruct((nrows, dp), _F32)
    mesh = plsc.VectorSubcoreMesh(core_axis_name="c", subcore_axis_name="s")

    @functools.partial(
        pl.kernel,
        out_type=sds,
        mesh=mesh,
        scratch_types=[
            pltpu.VMEM((rpw,), jnp.int32),
            pltpu.VMEM((grp, dp), _F32),
            pltpu.VMEM((grp, dp), _F32),
            pltpu.SemaphoreType.DMA,
            pltpu.SemaphoreType.DMA,
            pltpu.SemaphoreType.DMA,
            pltpu.SemaphoreType.DMA,
        ],
        compiler_params=pltpu.CompilerParams(use_tc_tiling_on_sc=False),
    )
    def k(table_h, idx_h, out_h, idxv, buf0, buf1, gs0, gs1, ws0, ws1):
        wid = lax.axis_index("s") * 2 + lax.axis_index("c")
        base = wid * rpw
        pltpu.sync_copy(idx_h.at[pl.ds(base, rpw)], idxv)
        pltpu.async_copy(table_h.at[idxv.at[pl.ds(0, grp)]], buf0, gs0)
        bufs = ((buf0, gs0, ws0), (buf1, gs1, ws1))

        def do_group(gg, s):
            buf, gs, ws = bufs[s]
            obuf, ogs, ows = bufs[1 - s]

            @pl.when(gg >= 1)
            def _():
                pltpu.make_async_copy(
                    obuf, out_h.at[pl.ds(base + (gg - 1) * grp, grp)],
                    ows).wait()

            @pl.when(gg + 1 < ngrp)
            def _():
                pltpu.async_copy(
                    table_h.at[idxv.at[pl.ds((gg + 1) * grp, grp)]],
                    obuf, ogs)

            pltpu.make_async_copy(
                table_h.at[idxv.at[pl.ds(gg * grp, grp)]], buf, gs).wait()
            pltpu.async_copy(buf, out_h.at[pl.ds(base + gg * grp, grp)], ws)

        def pair(t, _):
            do_group(2 * t, 0)
            do_group(2 * t + 1, 1)
            return 0

        lax.fori_loop(0, ngrp // 2, pair, 0)
        lbuf, _, lws = bufs[(ngrp - 1) % 2]
        pltpu.make_async_copy(
            lbuf, out_h.at[pl.ds(base + (ngrp - 1) * grp, grp)], lws).wait()

    return k(table, idxe)


def _edge_body(x_ref, xg_ref, wa_ref, wb_ref,
               fmax_ref, cs_ref, cq_ref, acc_s, acc_q):
    t = pl.program_id(0)
    nt = pl.num_programs(0)

    @pl.when(t == 0)
    def _():
        acc_s[...] = jnp.zeros_like(acc_s)
        acc_q[...] = jnp.zeros_like(acc_q)

    xi = x_ref[...]
    dn = (((1,), (0,)), ((), ()))
    fb = lax.dot_general(xi.astype(_BF16), wb_ref[...], dn,
                         preferred_element_type=_F32)
    fmax = None
    fsum = None
    fsq = None
    for kk in range(KNB):
        e1 = (xg_ref[kk] - xi).astype(_BF16)
        fk = lax.dot_general(e1, wa_ref[...], dn,
                             preferred_element_type=_F32) + fb
        if kk == 0:
            fmax, fsum, fsq = fk, fk, fk * fk
        else:
            fmax = jnp.maximum(fmax, fk)
            fsum = fsum + fk
            fsq = fsq + fk * fk
    fmax_ref[...] = fmax
    acc_s[...] += jnp.sum(fsum, axis=0, keepdims=True)
    acc_q[...] += jnp.sum(fsq, axis=0, keepdims=True)

    @pl.when(t == nt - 1)
    def _():
        cs_ref[...] = acc_s[...]
        cq_ref[...] = acc_q[...]


def _edge(x2d, xg3d, wa_bf, wb_bf):
    bn, d = x2d.shape
    c = wa_bf.shape[1]
    nt = bn // ROWT
    return pl.pallas_call(
        _edge_body,
        grid=(nt,),
        in_specs=[
            pl.BlockSpec((ROWT, d), lambda t: (t, 0)),
            pl.BlockSpec((KNB, ROWT, d), lambda t: (0, t, 0)),
            pl.BlockSpec((d, c), lambda t: (0, 0)),
            pl.BlockSpec((d, c), lambda t: (0, 0)),
        ],
        out_specs=[
            pl.BlockSpec((ROWT, c), lambda t: (t, 0)),
            pl.BlockSpec((1, c), lambda t: (0, 0)),
            pl.BlockSpec((1, c), lambda t: (0, 0)),
        ],
        out_shape=[
            jax.ShapeDtypeStruct((bn, c), _F32),
            jax.ShapeDtypeStruct((1, c), _F32),
            jax.ShapeDtypeStruct((1, c), _F32),
        ],
        scratch_shapes=[pltpu.VMEM((1, c), _F32), pltpu.VMEM((1, c), _F32)],
    )(x2d, xg3d, wa_bf, wb_bf)


def _bn_body(fmax_ref, cs_ref, cq_ref, g_ref, b_ref, o_ref):
    m = float(fmax_ref.shape[0] * KNB)
    mean = cs_ref[...] / m
    var = cq_ref[...] / m - mean * mean
    inv = lax.rsqrt(var + 1e-5)
    y = (fmax_ref[...] - mean) * inv * g_ref[...] + b_ref[...]
    o_ref[...] = jnp.where(y >= 0, y, 0.2 * y)


def _bn(fmax, cs, cq, gamma, beta):
    bn, c = fmax.shape
    return pl.pallas_call(
        _bn_body,
        out_shape=jax.ShapeDtypeStruct((bn, c), _F32),
    )(fmax, cs, cq, gamma, beta)


def _t3_body(x1_ref, x2_ref, x3_ref, x4_ref, w1_ref, w2_ref, w3_ref, w4_ref,
             g_ref, b_ref, o_ref, cs_ref, cq_ref):
    bb = pl.program_id(0)
    dn = (((1,), (0,)), ((), ()))
    y = (lax.dot_general(x1_ref[...].astype(_BF16), w1_ref[...], dn,
                         preferred_element_type=_F32)
         + lax.dot_general(x2_ref[...].astype(_BF16), w2_ref[...], dn,
                           preferred_element_type=_F32)
         + lax.dot_general(x3_ref[...].astype(_BF16), w3_ref[...], dn,
                           preferred_element_type=_F32)
         + lax.dot_general(x4_ref[...].astype(_BF16), w4_ref[...], dn,
                           preferred_element_type=_F32))

    @pl.when(bb == 0)
    def _():
        cs_ref[...] = jnp.zeros_like(cs_ref)
        cq_ref[...] = jnp.zeros_like(cq_ref)

    cs_ref[...] += jnp.sum(y, axis=0, keepdims=True)
    cq_ref[...] += jnp.sum(y * y, axis=0, keepdims=True)
    o_ref[pl.ds(bb, 1), :] = jnp.max(y, axis=0, keepdims=True)

    @pl.when(bb == NBATCH - 1)
    def _():
        m = float(NBATCH * NPTS)
        mean = cs_ref[...] / m
        var = cq_ref[...] / m - mean * mean
        inv = lax.rsqrt(var + 1e-5)
        v = (o_ref[...] - mean) * inv * g_ref[...] + b_ref[...]
        o_ref[...] = jnp.where(v >= 0, v, 0.2 * v)


def _t3(x1, x2, x3, x4, w_parts, gamma, beta):
    w1t, w2t, w3t, w4t = w_parts
    cdim = w1t.shape[1]
    specs = []
    for xin in (x1, x2, x3, x4):
        cc = xin.shape[1]
        specs.append(pl.BlockSpec((NPTS, cc), lambda b: (b, 0)))
    for wt in (w1t, w2t, w3t, w4t):
        dd = wt.shape[0]
        specs.append(pl.BlockSpec((dd, cdim), lambda b: (0, 0)))
    specs.append(pl.BlockSpec((1, cdim), lambda b: (0, 0)))
    specs.append(pl.BlockSpec((1, cdim), lambda b: (0, 0)))
    return pl.pallas_call(
        _t3_body,
        grid=(NBATCH,),
        in_specs=specs,
        out_specs=pl.BlockSpec((NBATCH, cdim), lambda b: (0, 0)),
        out_shape=jax.ShapeDtypeStruct((NBATCH, cdim), _F32),
        scratch_shapes=[pltpu.VMEM((1, cdim), _F32),
                        pltpu.VMEM((1, cdim), _F32)],
    )(x1, x2, x3, x4, w1t, w2t, w3t, w4t, gamma, beta)


def _run_block(x3d, w, gamma, beta):
    bq, nq, d = x3d.shape
    dp = 16 if d < 16 else d
    if dp != d:
        x3d = jnp.pad(x3d, ((0, 0), (0, 0), (0, dp - d)))
    xt = jnp.transpose(x3d, (0, 2, 1))
    idx2 = _t1(x3d, xt)
    idxk = jnp.transpose(idx2, (1, 0)).reshape(-1)
    x2d = x3d.reshape(bq * nq, dp)
    xg = _sc_gather(x2d, idxk)
    wa_bf = jnp.pad(w[:, :d].T.astype(_BF16), ((0, dp - d), (0, 0)))
    wb_bf = jnp.pad(w[:, d:].T.astype(_BF16), ((0, dp - d), (0, 0)))
    fmax, cs, cq = _edge(x2d, xg.reshape(KNB, bq * nq, dp), wa_bf, wb_bf)
    return _bn(fmax, cs, cq, gamma.reshape(1, -1), beta.reshape(1, -1))


def kernel(x, fc1_w, fc2_w, fc3_w, fc4_w, fc5_w,
           bn1_g, bn1_b, bn2_g, bn2_b, bn3_g, bn3_b,
           bn4_g, bn4_b, bn5_g, bn5_b):
    bq, nq, _ = x.shape
    x1 = _run_block(x, fc1_w, bn1_g, bn1_b)
    x2 = _run_block(x1.reshape(bq, nq, -1), fc2_w, bn2_g, bn2_b)
    x3 = _run_block(x2.reshape(bq, nq, -1), fc3_w, bn3_g, bn3_b)
    x4 = _run_block(x3.reshape(bq, nq, -1), fc4_w, bn4_g, bn4_b)
    w5_parts = (fc5_w[:, :64].T.astype(_BF16),
                fc5_w[:, 64:128].T.astype(_BF16),
                fc5_w[:, 128:256].T.astype(_BF16),
                fc5_w[:, 256:].T.astype(_BF16))
    return _t3(x1, x2, x3, x4, w5_parts,
               bn5_g.reshape(1, -1), bn5_b.reshape(1, -1))
